# baseline (speedup 1.0000x reference)
import functools

import jax
import jax.numpy as jnp
from jax import lax
from jax.experimental import pallas as pl
from jax.experimental.pallas import tpu as pltpu
from jax.experimental.pallas import tpu_sc as plsc

NC = 2
NS = 16
L = 16
NW = NC * NS

B = 16384
K = 28
CHUNK = B // NW
SUB = 128
NSUB = CHUNK // SUB
RN = 16
NRND = CHUNK // RN
KH = K // 2

_mesh = plsc.VectorSubcoreMesh(core_axis_name="c", subcore_axis_name="s")


@functools.partial(
    pl.kernel,
    mesh=_mesh,
    out_type=(jax.ShapeDtypeStruct((B, L), jnp.float32),
              jax.ShapeDtypeStruct((B,), jnp.float32)),
    scratch_types=[
        pltpu.VMEM((K, NSUB, SUB), jnp.int32),
        pltpu.VMEM((K, RN), jnp.int32),
        pltpu.VMEM((K, NSUB, SUB), jnp.float32),
        pltpu.VMEM((KH, RN, 128), jnp.float32),
        pltpu.VMEM((RN, L), jnp.float32),
        pltpu.VMEM((CHUNK,), jnp.float32),
        pltpu.SemaphoreType.DMA,
        pltpu.SemaphoreType.DMA,
    ],
)
def _fm_sc_kernel(idx_hbm, user_t, item_t, meta_t, lin_u, lin_i, lin_m,
                  t_hbm, lin_hbm, idx_v, blk_v, linv, rows, t_v,
                  lsum_v, sem_e, sem_l):
    wid = lax.axis_index("s") * NC + lax.axis_index("c")
    base = wid * CHUNK

    pltpu.sync_copy(idx_hbm.at[wid], idx_v)

    tables = [user_t, item_t] + [meta_t] * (K - 2)
    lins = [lin_u, lin_i] + [lin_m] * (K - 2)

    lin_cps = []
    for c in range(NSUB):
        for k in range(K):
            lin_cps.append(
                pltpu.async_copy(lins[k].at[idx_v.at[k, c]], linv.at[k, c], sem_l))

    def round_body(r, _):
        c = r // (SUB // RN)
        rr = r % (SUB // RN)

        ovecs = []
        for k in range(KH):
            raw = idx_v[k, c, pl.ds(rr * RN, RN)]
            blk_v[k, :] = jnp.right_shift(raw, 3)
            ovecs.append((raw & 7) * L)
        cps = [pltpu.async_copy(tables[k].at[blk_v.at[k]],
                                rows.at[k], sem_e)
               for k in range(KH)]
        for cp in cps:
            cp.wait()
        acc_s, acc_q = [], []
        for m in range(RN):
            o0 = ovecs[0][m]
            s = rows[0, m, pl.ds(o0, L)]
            q = s * s
            for k in range(1, KH):
                o = ovecs[k][m]
                v = rows[k, m, pl.ds(o, L)]
                s = s + v
                q = q + v * v
            acc_s.append(s)
            acc_q.append(q)

        ovecs = []
        for k in range(KH, K):
            raw = idx_v[k, c, pl.ds(rr * RN, RN)]
            blk_v[k, :] = jnp.right_shift(raw, 3)
            ovecs.append((raw & 7) * L)
        cps = [pltpu.async_copy(tables[k].at[blk_v.at[k]],
                                rows.at[k - KH], sem_e)
               for k in range(KH, K)]
        for cp in cps:
            cp.wait()
        for m in range(RN):
            s, q = acc_s[m], acc_q[m]
            for k in range(KH, K):
                o = ovecs[k - KH][m]
                v = rows[k - KH, m, pl.ds(o, L)]
                s = s + v
                q = q + v * v
            t_v[m, :] = s * s - q
        pltpu.sync_copy(t_v, t_hbm.at[pl.ds(base + r * RN, RN)])
        return 0

    lax.fori_loop(0, NRND, round_body, 0)

    for cp in lin_cps:
        cp.wait()

    for c in range(NSUB):
        def lbody(j, _, c=c):
            lacc = linv[0, c, pl.ds(j * L, L)]
            for k in range(1, K):
                lacc = lacc + linv[k, c, pl.ds(j * L, L)]
            lsum_v[pl.ds(c * SUB + j * L, L)] = lacc
            return 0

        lax.fori_loop(0, SUB // L, lbody, 0)

    pltpu.sync_copy(lsum_v, lin_hbm.at[pl.ds(base, CHUNK)])


def _tc_body(t_ref, lin_ref, o_ref):
    z = lin_ref[:] + 0.5 * jnp.sum(t_ref[:], axis=1)
    o_ref[:] = 1.0 / (1.0 + jnp.exp(-z))


_tc_epilogue = pl.pallas_call(
    _tc_body,
    out_shape=jax.ShapeDtypeStruct((B,), jnp.float32),
)


def kernel(user, item, metadata, user_table, item_table, meta_tables,
           lin_user_table, lin_item_table, lin_meta_tables):
    M, Vm, F = meta_tables.shape
    offs = (jnp.arange(M, dtype=jnp.int32) * Vm)[:, None]
    idx_all = jnp.concatenate(
        [user[None, :].astype(jnp.int32),
         item[None, :].astype(jnp.int32),
         metadata.T.astype(jnp.int32) + offs], axis=0)
    idx_r = (idx_all.reshape(K, NW, CHUNK).transpose(1, 0, 2)
             .reshape(NW, K, NSUB, SUB))
    u128 = user_table.reshape(user_table.shape[0] * F // 128, 128)
    i128 = item_table.reshape(item_table.shape[0] * F // 128, 128)
    m128 = meta_tables.reshape(M * Vm * F // 128, 128)
    t_out, lin_out = _fm_sc_kernel(idx_r, u128, i128, m128,
                                   lin_user_table.reshape(-1),
                                   lin_item_table.reshape(-1),
                                   lin_meta_tables.reshape(-1))
    return _tc_epilogue(t_out, lin_out)
